# baseline (speedup 1.0000x reference)
import functools

import jax
import jax.numpy as jnp
from jax import lax
from jax.experimental import pallas as pl
from jax.experimental.pallas import tpu as pltpu
from jax.experimental.pallas import tpu_sc as plsc

N = 10000
NP = 10240
E = 160000
EP = 163840
NTILE = 16
EW = 64
EC = EP // NTILE // EW
NSTAGE = 4
RPT = NP // NTILE


def _mesh():
  return plsc.VectorSubcoreMesh(
      core_axis_name="c", subcore_axis_name="s", num_cores=2, num_subcores=16)


def _deg_partials(dst32):
  nrow = EP // 32 // 128

  @functools.partial(
      pl.kernel,
      out_type=jax.ShapeDtypeStruct((32, NP), jnp.float32),
      mesh=_mesh(),
      compiler_params=pltpu.CompilerParams(needs_layout_passes=False),
      scratch_types=[
          pltpu.VMEM((nrow, 128), jnp.int32),
          pltpu.VMEM((NP,), jnp.float32),
      ],
  )
  def k(dst_hbm, degp_hbm, idxv, degl):
    c = lax.axis_index("c")
    s = lax.axis_index("s")
    w = c * NTILE + s

    def zrow(i, carry):
      degl[pl.ds(i * 16, 16)] = jnp.zeros((16,), jnp.float32)
      return carry

    lax.fori_loop(0, NP // 16, zrow, 0)
    pltpu.sync_copy(dst_hbm.at[w], idxv)
    ones = jnp.full((16,), 1.0, jnp.float32)

    def body(r, carry):
      for q in range(8):
        idx = idxv[r, pl.ds(q * 16, 16)]
        plsc.addupdate_scatter(degl, [idx], ones)
      return carry

    lax.fori_loop(0, nrow, body, 0)
    pltpu.sync_copy(degl, degp_hbm.at[w])

  return k(dst32)


def _dinv(degp_t):
  RT = 2048

  def body(d_ref, o_ref):
    s = jnp.sum(d_ref[...], axis=1, keepdims=True) + 1.0
    o_ref[...] = lax.rsqrt(s)

  return pl.pallas_call(
      body,
      grid=(NP // RT,),
      in_specs=[pl.BlockSpec((RT, 32), lambda i: (i, 0))],
      out_specs=pl.BlockSpec((RT, 1), lambda i: (i, 0)),
      out_shape=jax.ShapeDtypeStruct((NP, 1), jnp.float32),
  )(degp_t)


def _pre(xcat, w, dinv, h):
  K = xcat.shape[1]
  RT = 512

  def body(x_ref, w_ref, d_ref, y0_ref, y1_ref):
    y = jnp.dot(x_ref[...], w_ref[...], preferred_element_type=jnp.float32)
    y = y * d_ref[...]
    y0_ref[...] = y[:, :h]
    y1_ref[...] = y[:, h:]

  return pl.pallas_call(
      body,
      grid=(NP // RT,),
      in_specs=[
          pl.BlockSpec((RT, K), lambda i: (i, 0)),
          pl.BlockSpec((K, 2 * h), lambda i: (0, 0)),
          pl.BlockSpec((RT, 1), lambda i: (i, 0)),
      ],
      out_specs=[pl.BlockSpec((RT, h), lambda i: (i, 0))] * 2,
      out_shape=[jax.ShapeDtypeStruct((NP, h), jnp.float32)] * 2,
  )(xcat, w, dinv)


def _agg(y0, y1, src3, dst3, h):
  NB = 4
  CHS = EC // NSTAGE

  @functools.partial(
      pl.kernel,
      out_type=[jax.ShapeDtypeStruct((NP, h), jnp.float32)] * 2,
      mesh=_mesh(),
      compiler_params=pltpu.CompilerParams(needs_layout_passes=False),
      scratch_types=[
          pltpu.VMEM((CHS, EW), jnp.int32),
          pltpu.VMEM((CHS, EW), jnp.int32),
          pltpu.VMEM((EW, h), jnp.float32),
          pltpu.VMEM((EW, h), jnp.float32),
          pltpu.VMEM((EW, h), jnp.float32),
          pltpu.VMEM((EW, h), jnp.float32),
          pltpu.VMEM_SHARED((NP, h), jnp.float32),
          pltpu.SemaphoreType.DMA,
          pltpu.SemaphoreType.DMA,
          pltpu.SemaphoreType.DMA,
          pltpu.SemaphoreType.DMA,
      ],
  )
  def k(y0h, y1h, src_h, dst_h, a0h, a1h, sv, dv, gb0, gb1, gb2, gb3, acc,
        sem0, sem1, sem2, sem3):
    c = lax.axis_index("c")
    s = lax.axis_index("s")
    gbs = (gb0, gb1, gb2, gb3)
    sems = (sem0, sem1, sem2, sem3)

    pltpu.sync_copy(src_h.at[s, pl.ds(0, CHS)], sv)
    pltpu.sync_copy(dst_h.at[s, pl.ds(0, CHS)], dv)

    def zrow(i, carry):
      for q in range(h // 16):
        gb0[i, pl.ds(q * 16, 16)] = jnp.zeros((16,), jnp.float32)
      return carry

    lax.fori_loop(0, EW, zrow, 0)
    for r in range(RPT // EW):
      pltpu.async_copy(gb0, acc.at[pl.ds(s * RPT + r * EW, EW)],
                       sems[r % NB])
    for r in range(RPT // EW):
      pltpu.make_async_copy(
          gb0, acc.at[pl.ds(s * RPT + r * EW, EW)], sems[r % NB]).wait()
    plsc.subcore_barrier()

    def edge_chunks(yh):
      for stage in range(NSTAGE):
        if stage:
          pltpu.sync_copy(src_h.at[s, pl.ds(stage * CHS, CHS)], sv)
          pltpu.sync_copy(dst_h.at[s, pl.ds(stage * CHS, CHS)], dv)
        for b in range(NB):
          pltpu.async_copy(yh.at[sv.at[b]], gbs[b], sems[b])

        def group(g, carry):
          for b in range(NB):
            j = g * NB + b
            pltpu.make_async_copy(yh.at[sv.at[j]], gbs[b], sems[b]).wait()
            pltpu.sync_copy(gbs[b], acc.at[dv.at[j]], add=True)
            pltpu.async_copy(yh.at[sv.at[j + NB]], gbs[b], sems[b])
          return carry

        lax.fori_loop(0, CHS // NB - 1, group, 0)
        for b in range(NB):
          j = CHS - NB + b
          pltpu.make_async_copy(yh.at[sv.at[j]], gbs[b], sems[b]).wait()
          pltpu.sync_copy(gbs[b], acc.at[dv.at[j]], add=True)

    @pl.when(c == 0)
    def _():
      edge_chunks(y0h)

    @pl.when(c == 1)
    def _():
      edge_chunks(y1h)

    plsc.subcore_barrier()

    @pl.when(c == 0)
    def _():
      pltpu.sync_copy(acc.at[pl.ds(s * RPT, RPT)], a0h.at[pl.ds(s * RPT, RPT)])

    @pl.when(c == 1)
    def _():
      pltpu.sync_copy(acc.at[pl.ds(s * RPT, RPT)], a1h.at[pl.ds(s * RPT, RPT)])

  return k(y0, y1, src3, dst3)


def _post(a0, a1, y0, y1, dinv, bias, prev, h, relu):
  RT = 1024
  has_prev = prev is not None

  def body(*refs):
    if has_prev:
      a0r, a1r, y0r, y1r, dr, br, pr, o_ref = refs
    else:
      a0r, a1r, y0r, y1r, dr, br, o_ref = refs
    d = dr[...]
    o0 = d * (a0r[...] + y0r[...])
    o1 = d * (a1r[...] + y1r[...])
    o = jnp.concatenate([o0, o1], axis=1) + br[...]
    if has_prev:
      o = o + pr[...]
    if relu:
      o = jnp.maximum(o, 0.0)
    o_ref[...] = o

  in_specs = [
      pl.BlockSpec((RT, h), lambda i: (i, 0)),
      pl.BlockSpec((RT, h), lambda i: (i, 0)),
      pl.BlockSpec((RT, h), lambda i: (i, 0)),
      pl.BlockSpec((RT, h), lambda i: (i, 0)),
      pl.BlockSpec((RT, 1), lambda i: (i, 0)),
      pl.BlockSpec((1, 2 * h), lambda i: (0, 0)),
  ]
  args = [a0, a1, y0, y1, dinv, bias]
  if has_prev:
    in_specs.append(pl.BlockSpec((RT, 2 * h), lambda i: (i, 0)))
    args.append(prev)
  return pl.pallas_call(
      body,
      grid=(NP // RT,),
      in_specs=in_specs,
      out_specs=pl.BlockSpec((RT, 2 * h), lambda i: (i, 0)),
      out_shape=jax.ShapeDtypeStruct((NP, 2 * h), jnp.float32),
  )(*args)


def _layer(xcat, W, b, dinv, src3, dst3, prev=None, relu=False):
  dout = W.shape[1]
  h = dout // 2
  y0, y1 = _pre(xcat, W, dinv, h)
  a0, a1 = _agg(y0, y1, src3, dst3, h)
  return _post(a0, a1, y0, y1, dinv, b.reshape(1, dout), prev, h, relu)


def kernel(x, edge_index, Ws, bs):
  xp = jnp.pad(x.astype(jnp.float32), ((0, NP - N), (0, 0)))
  src = edge_index[0].astype(jnp.int32)
  dst = edge_index[1].astype(jnp.int32)
  pad = jnp.full((EP - E,), NP - 1, jnp.int32)
  src = jnp.concatenate([src, pad])
  dst = jnp.concatenate([dst, pad])
  src3 = src.reshape(NTILE, EC, EW)
  dst3 = dst.reshape(NTILE, EC, EW)
  dst32 = dst.reshape(32, EP // 32 // 128, 128)

  degp = _deg_partials(dst32)
  dinv = _dinv(degp.T)

  h = _layer(xp, Ws[0], bs[0], dinv, src3, dst3)
  xcat = h
  xblk = h
  for i in range(5):
    xblk = _layer(xcat, Ws[1 + i], bs[1 + i], dinv, src3, dst3,
                  prev=xblk, relu=True)
    xcat = jnp.concatenate([xcat, xblk], axis=1)
  g = _layer(xcat, Ws[6], bs[6], dinv, src3, dst3)
  xcat = g
  xblk = g
  for i in range(5):
    xblk = _layer(xcat, Ws[7 + i], bs[7 + i], dinv, src3, dst3,
                  prev=xblk, relu=True)
    xcat = jnp.concatenate([xcat, xblk], axis=1)
  dlast = Ws[12].shape[1]
  wlast = jnp.pad(Ws[12], ((0, 0), (0, 256 - dlast)))
  blast = jnp.pad(bs[12], (0, 256 - dlast))
  out = _layer(xcat, wlast, blast, dinv, src3, dst3)
  return out[:N, :dlast]


# speedup vs baseline: 1.0464x; 1.0464x over previous
import functools

import jax
import jax.numpy as jnp
from jax import lax
from jax.experimental import pallas as pl
from jax.experimental.pallas import tpu as pltpu
from jax.experimental.pallas import tpu_sc as plsc

N = 10000
NP = 10240
E = 160000
EP = 163840
NTILE = 16
NCHUNK = EP // NTILE // 128
CH = 128
RPT = NP // NTILE


def _mesh():
  return plsc.VectorSubcoreMesh(
      core_axis_name="c", subcore_axis_name="s", num_cores=2, num_subcores=16)


def _deg_partials(dst32):
  nrow = EP // 32 // 128

  @functools.partial(
      pl.kernel,
      out_type=jax.ShapeDtypeStruct((32, NP), jnp.float32),
      mesh=_mesh(),
      compiler_params=pltpu.CompilerParams(needs_layout_passes=False),
      scratch_types=[
          pltpu.VMEM((nrow, 128), jnp.int32),
          pltpu.VMEM((NP,), jnp.float32),
      ],
  )
  def k(dst_hbm, degp_hbm, idxv, degl):
    c = lax.axis_index("c")
    s = lax.axis_index("s")
    w = c * NTILE + s

    def zrow(i, carry):
      degl[pl.ds(i * 16, 16)] = jnp.zeros((16,), jnp.float32)
      return carry

    lax.fori_loop(0, NP // 16, zrow, 0)
    pltpu.sync_copy(dst_hbm.at[w], idxv)
    ones = jnp.full((16,), 1.0, jnp.float32)

    def body(r, carry):
      for q in range(8):
        idx = idxv[r, pl.ds(q * 16, 16)]
        plsc.addupdate_scatter(degl, [idx], ones)
      return carry

    lax.fori_loop(0, nrow, body, 0)
    pltpu.sync_copy(degl, degp_hbm.at[w])

  return k(dst32)


def _dinv(degp_t):
  RT = 2048

  def body(d_ref, o_ref):
    s = jnp.sum(d_ref[...], axis=1, keepdims=True) + 1.0
    o_ref[...] = lax.rsqrt(s)

  return pl.pallas_call(
      body,
      grid=(NP // RT,),
      in_specs=[pl.BlockSpec((RT, 32), lambda i: (i, 0))],
      out_specs=pl.BlockSpec((RT, 1), lambda i: (i, 0)),
      out_shape=jax.ShapeDtypeStruct((NP, 1), jnp.float32),
  )(degp_t)


def _matsum(pieces, w, dout):
  n = len(pieces)
  K = pieces[0].shape[1]
  RT = 512

  def body(*refs):
    x_refs = refs[:n]
    w_ref = refs[n]
    o_ref = refs[n + 1]
    acc = jnp.dot(x_refs[0][...], w_ref[pl.ds(0, K), :],
                  preferred_element_type=jnp.float32)
    for p in range(1, n):
      acc = acc + jnp.dot(x_refs[p][...], w_ref[pl.ds(p * K, K), :],
                          preferred_element_type=jnp.float32)
    o_ref[...] = acc

  return pl.pallas_call(
      body,
      grid=(NP // RT,),
      in_specs=[pl.BlockSpec((RT, K), lambda i: (i, 0))] * n
      + [pl.BlockSpec((n * K, dout), lambda i: (0, 0))],
      out_specs=pl.BlockSpec((RT, dout), lambda i: (i, 0)),
      out_shape=jax.ShapeDtypeStruct((NP, dout), jnp.float32),
  )(*pieces, w)


def _pre(partial, xlast, wlast, dinv, h):
  K = xlast.shape[1]
  RT = 512
  has_partial = partial is not None

  def body(*refs):
    if has_partial:
      p_ref, x_ref, w_ref, d_ref, y0_ref, y1_ref = refs
    else:
      x_ref, w_ref, d_ref, y0_ref, y1_ref = refs
    y = jnp.dot(x_ref[...], w_ref[...], preferred_element_type=jnp.float32)
    if has_partial:
      y = y + p_ref[...]
    y = y * d_ref[...]
    y0_ref[...] = y[:, :h]
    y1_ref[...] = y[:, h:]

  in_specs = []
  args = []
  if has_partial:
    in_specs.append(pl.BlockSpec((RT, 2 * h), lambda i: (i, 0)))
    args.append(partial)
  in_specs += [
      pl.BlockSpec((RT, K), lambda i: (i, 0)),
      pl.BlockSpec((K, 2 * h), lambda i: (0, 0)),
      pl.BlockSpec((RT, 1), lambda i: (i, 0)),
  ]
  args += [xlast, wlast, dinv]
  return pl.pallas_call(
      body,
      grid=(NP // RT,),
      in_specs=in_specs,
      out_specs=[pl.BlockSpec((RT, h), lambda i: (i, 0))] * 2,
      out_shape=[jax.ShapeDtypeStruct((NP, h), jnp.float32)] * 2,
  )(*args)


def _agg(y0, y1, src3, dst3, h):
  HALF = NCHUNK // 2

  @functools.partial(
      pl.kernel,
      out_type=[jax.ShapeDtypeStruct((NP, h), jnp.float32)] * 2,
      mesh=_mesh(),
      compiler_params=pltpu.CompilerParams(needs_layout_passes=False),
      scratch_types=[
          pltpu.VMEM((HALF, CH), jnp.int32),
          pltpu.VMEM((HALF, CH), jnp.int32),
          pltpu.VMEM((CH, h), jnp.float32),
          pltpu.VMEM((CH, h), jnp.float32),
          pltpu.VMEM_SHARED((NP, h), jnp.float32),
          pltpu.SemaphoreType.DMA,
          pltpu.SemaphoreType.DMA,
      ],
  )
  def k(y0h, y1h, src_h, dst_h, a0h, a1h, sv, dv, gb0, gb1, acc,
        sem0, sem1):
    c = lax.axis_index("c")
    s = lax.axis_index("s")
    gbs = (gb0, gb1)
    sems = (sem0, sem1)

    pltpu.sync_copy(src_h.at[s, pl.ds(0, HALF)], sv)
    pltpu.sync_copy(dst_h.at[s, pl.ds(0, HALF)], dv)

    def zrow(i, carry):
      for q in range(h // 16):
        gb0[i, pl.ds(q * 16, 16)] = jnp.zeros((16,), jnp.float32)
      return carry

    lax.fori_loop(0, CH, zrow, 0)
    for r in range(RPT // CH):
      pltpu.async_copy(gb0, acc.at[pl.ds(s * RPT + r * CH, CH)], sems[r % 2])
    for r in range(RPT // CH):
      pltpu.make_async_copy(
          gb0, acc.at[pl.ds(s * RPT + r * CH, CH)], sems[r % 2]).wait()
    plsc.subcore_barrier()

    def edge_chunks(yh):
      for half in range(2):
        if half:
          pltpu.sync_copy(src_h.at[s, pl.ds(HALF, HALF)], sv)
          pltpu.sync_copy(dst_h.at[s, pl.ds(HALF, HALF)], dv)
        for b in range(2):
          pltpu.async_copy(yh.at[sv.at[b]], gbs[b], sems[b])

        def group(g, carry):
          for b in range(2):
            j = g * 2 + b
            pltpu.make_async_copy(yh.at[sv.at[j]], gbs[b], sems[b]).wait()
            pltpu.sync_copy(gbs[b], acc.at[dv.at[j]], add=True)
            pltpu.async_copy(yh.at[sv.at[j + 2]], gbs[b], sems[b])
          return carry

        lax.fori_loop(0, HALF // 2 - 1, group, 0)
        for b in range(2):
          j = HALF - 2 + b
          pltpu.make_async_copy(yh.at[sv.at[j]], gbs[b], sems[b]).wait()
          pltpu.sync_copy(gbs[b], acc.at[dv.at[j]], add=True)

    @pl.when(c == 0)
    def _():
      edge_chunks(y0h)

    @pl.when(c == 1)
    def _():
      edge_chunks(y1h)

    plsc.subcore_barrier()

    @pl.when(c == 0)
    def _():
      pltpu.sync_copy(acc.at[pl.ds(s * RPT, RPT)], a0h.at[pl.ds(s * RPT, RPT)])

    @pl.when(c == 1)
    def _():
      pltpu.sync_copy(acc.at[pl.ds(s * RPT, RPT)], a1h.at[pl.ds(s * RPT, RPT)])

  return k(y0, y1, src3, dst3)


def _post(a0, a1, y0, y1, dinv, bias, prev, h, relu):
  RT = 1024
  has_prev = prev is not None

  def body(*refs):
    if has_prev:
      a0r, a1r, y0r, y1r, dr, br, pr, o_ref = refs
    else:
      a0r, a1r, y0r, y1r, dr, br, o_ref = refs
    d = dr[...]
    o0 = d * (a0r[...] + y0r[...])
    o1 = d * (a1r[...] + y1r[...])
    o = jnp.concatenate([o0, o1], axis=1) + br[...]
    if has_prev:
      o = o + pr[...]
    if relu:
      o = jnp.maximum(o, 0.0)
    o_ref[...] = o

  in_specs = [
      pl.BlockSpec((RT, h), lambda i: (i, 0)),
      pl.BlockSpec((RT, h), lambda i: (i, 0)),
      pl.BlockSpec((RT, h), lambda i: (i, 0)),
      pl.BlockSpec((RT, h), lambda i: (i, 0)),
      pl.BlockSpec((RT, 1), lambda i: (i, 0)),
      pl.BlockSpec((1, 2 * h), lambda i: (0, 0)),
  ]
  args = [a0, a1, y0, y1, dinv, bias]
  if has_prev:
    in_specs.append(pl.BlockSpec((RT, 2 * h), lambda i: (i, 0)))
    args.append(prev)
  return pl.pallas_call(
      body,
      grid=(NP // RT,),
      in_specs=in_specs,
      out_specs=pl.BlockSpec((RT, 2 * h), lambda i: (i, 0)),
      out_shape=jax.ShapeDtypeStruct((NP, 2 * h), jnp.float32),
  )(*args)


def _layer(pieces, W, b, dinv, src3, dst3, prev=None, relu=False):
  dout = W.shape[1]
  h = dout // 2
  n = len(pieces)
  K = pieces[0].shape[1]
  if n > 1:
    partial = _matsum(pieces[:-1], W[: (n - 1) * K], dout)
  else:
    partial = None
  y0, y1 = _pre(partial, pieces[-1], W[(n - 1) * K:], dinv, h)
  a0, a1 = _agg(y0, y1, src3, dst3, h)
  return _post(a0, a1, y0, y1, dinv, b.reshape(1, dout), prev, h, relu)


def kernel(x, edge_index, Ws, bs):
  xp = jnp.pad(x.astype(jnp.float32), ((0, NP - N), (0, 0)))
  src = edge_index[0].astype(jnp.int32)
  dst = edge_index[1].astype(jnp.int32)
  pad = jnp.full((EP - E,), NP - 1, jnp.int32)
  src = jnp.concatenate([src, pad])
  dst = jnp.concatenate([dst, pad])
  src3 = src.reshape(NTILE, NCHUNK, CH)
  dst3 = dst.reshape(NTILE, NCHUNK, CH)
  dst32 = dst.reshape(32, EP // 32 // 128, 128)

  degp = _deg_partials(dst32)
  dinv = _dinv(degp.T)

  h1 = _layer([xp], Ws[0], bs[0], dinv, src3, dst3)
  pieces = [h1]
  xblk = h1
  for i in range(5):
    xblk = _layer(pieces, Ws[1 + i], bs[1 + i], dinv, src3, dst3,
                  prev=xblk, relu=True)
    pieces = pieces + [xblk]
  g = _layer(pieces, Ws[6], bs[6], dinv, src3, dst3)
  pieces = [g]
  xblk = g
  for i in range(5):
    xblk = _layer(pieces, Ws[7 + i], bs[7 + i], dinv, src3, dst3,
                  prev=xblk, relu=True)
    pieces = pieces + [xblk]
  dlast = Ws[12].shape[1]
  wlast = jnp.pad(Ws[12], ((0, 0), (0, 256 - dlast)))
  blast = jnp.pad(bs[12], (0, 256 - dlast))
  out = _layer(pieces, wlast, blast, dinv, src3, dst3)
  return out[:N, :dlast]


# speedup vs baseline: 1.1639x; 1.1123x over previous
import functools

import jax
import jax.numpy as jnp
from jax import lax
from jax.experimental import pallas as pl
from jax.experimental.pallas import tpu as pltpu
from jax.experimental.pallas import tpu_sc as plsc

N = 10000
NP = 10240
E = 160000
EP = 163840
NTILE = 16
NCHUNK = EP // NTILE // 128
CH = 128
RPT = NP // NTILE


def _mesh():
  return plsc.VectorSubcoreMesh(
      core_axis_name="c", subcore_axis_name="s", num_cores=2, num_subcores=16)


def _deg_partials(dst32):
  nrow = EP // 32 // 128

  @functools.partial(
      pl.kernel,
      out_type=jax.ShapeDtypeStruct((32, NP), jnp.float32),
      mesh=_mesh(),
      compiler_params=pltpu.CompilerParams(needs_layout_passes=False),
      scratch_types=[
          pltpu.VMEM((nrow, 128), jnp.int32),
          pltpu.VMEM((NP,), jnp.float32),
      ],
  )
  def k(dst_hbm, degp_hbm, idxv, degl):
    c = lax.axis_index("c")
    s = lax.axis_index("s")
    w = c * NTILE + s

    def zrow(i, carry):
      degl[pl.ds(i * 16, 16)] = jnp.zeros((16,), jnp.float32)
      return carry

    lax.fori_loop(0, NP // 16, zrow, 0)
    pltpu.sync_copy(dst_hbm.at[w], idxv)
    ones = jnp.full((16,), 1.0, jnp.float32)

    def body(r, carry):
      for q in range(8):
        idx = idxv[r, pl.ds(q * 16, 16)]
        plsc.addupdate_scatter(degl, [idx], ones)
      return carry

    lax.fori_loop(0, nrow, body, 0)
    pltpu.sync_copy(degl, degp_hbm.at[w])

  return k(dst32)


def _dinv(degp_t):
  RT = 2048

  def body(d_ref, o_ref):
    s = jnp.sum(d_ref[...], axis=1, keepdims=True) + 1.0
    o_ref[...] = lax.rsqrt(s)

  return pl.pallas_call(
      body,
      grid=(NP // RT,),
      in_specs=[pl.BlockSpec((RT, 32), lambda i: (i, 0))],
      out_specs=pl.BlockSpec((RT, 1), lambda i: (i, 0)),
      out_shape=jax.ShapeDtypeStruct((NP, 1), jnp.float32),
  )(degp_t)


def _pre(x, w, dinv, h):
  K = x.shape[1]
  RT = 512

  def body(x_ref, w_ref, d_ref, y0_ref, y1_ref):
    y = jnp.dot(x_ref[...], w_ref[...], preferred_element_type=jnp.float32)
    y = y * d_ref[...]
    y0_ref[...] = y[:, :h]
    y1_ref[...] = y[:, h:]

  return pl.pallas_call(
      body,
      grid=(NP // RT,),
      in_specs=[
          pl.BlockSpec((RT, K), lambda i: (i, 0)),
          pl.BlockSpec((K, 2 * h), lambda i: (0, 0)),
          pl.BlockSpec((RT, 1), lambda i: (i, 0)),
      ],
      out_specs=[pl.BlockSpec((RT, h), lambda i: (i, 0))] * 2,
      out_shape=[jax.ShapeDtypeStruct((NP, h), jnp.float32)] * 2,
  )(x, w, dinv)


def _agg(y0, y1, src3, dst3, h):
  HALF = NCHUNK // 2

  @functools.partial(
      pl.kernel,
      out_type=[jax.ShapeDtypeStruct((NP, h), jnp.float32)] * 2,
      mesh=_mesh(),
      compiler_params=pltpu.CompilerParams(needs_layout_passes=False),
      scratch_types=[
          pltpu.VMEM((HALF, CH), jnp.int32),
          pltpu.VMEM((HALF, CH), jnp.int32),
          pltpu.VMEM((CH, h), jnp.float32),
          pltpu.VMEM((CH, h), jnp.float32),
          pltpu.VMEM_SHARED((NP, h), jnp.float32),
          pltpu.SemaphoreType.DMA,
          pltpu.SemaphoreType.DMA,
      ],
  )
  def k(y0h, y1h, src_h, dst_h, a0h, a1h, sv, dv, gb0, gb1, acc,
        sem0, sem1):
    c = lax.axis_index("c")
    s = lax.axis_index("s")
    gbs = (gb0, gb1)
    sems = (sem0, sem1)

    pltpu.sync_copy(src_h.at[s, pl.ds(0, HALF)], sv)
    pltpu.sync_copy(dst_h.at[s, pl.ds(0, HALF)], dv)

    def zrow(i, carry):
      for q in range(h // 16):
        gb0[i, pl.ds(q * 16, 16)] = jnp.zeros((16,), jnp.float32)
      return carry

    lax.fori_loop(0, CH, zrow, 0)
    for r in range(RPT // CH):
      pltpu.async_copy(gb0, acc.at[pl.ds(s * RPT + r * CH, CH)], sems[r % 2])
    for r in range(RPT // CH):
      pltpu.make_async_copy(
          gb0, acc.at[pl.ds(s * RPT + r * CH, CH)], sems[r % 2]).wait()
    plsc.subcore_barrier()

    def edge_chunks(yh):
      for half in range(2):
        if half:
          pltpu.sync_copy(src_h.at[s, pl.ds(HALF, HALF)], sv)
          pltpu.sync_copy(dst_h.at[s, pl.ds(HALF, HALF)], dv)
        for b in range(2):
          pltpu.async_copy(yh.at[sv.at[b]], gbs[b], sems[b])

        def group(g, carry):
          for b in range(2):
            j = g * 2 + b
            pltpu.make_async_copy(yh.at[sv.at[j]], gbs[b], sems[b]).wait()
            pltpu.sync_copy(gbs[b], acc.at[dv.at[j]], add=True)
            pltpu.async_copy(yh.at[sv.at[j + 2]], gbs[b], sems[b])
          return carry

        lax.fori_loop(0, HALF // 2 - 1, group, 0)
        for b in range(2):
          j = HALF - 2 + b
          pltpu.make_async_copy(yh.at[sv.at[j]], gbs[b], sems[b]).wait()
          pltpu.sync_copy(gbs[b], acc.at[dv.at[j]], add=True)

    @pl.when(c == 0)
    def _():
      edge_chunks(y0h)

    @pl.when(c == 1)
    def _():
      edge_chunks(y1h)

    plsc.subcore_barrier()

    @pl.when(c == 0)
    def _():
      pltpu.sync_copy(acc.at[pl.ds(s * RPT, RPT)], a0h.at[pl.ds(s * RPT, RPT)])

    @pl.when(c == 1)
    def _():
      pltpu.sync_copy(acc.at[pl.ds(s * RPT, RPT)], a1h.at[pl.ds(s * RPT, RPT)])

  return k(y0, y1, src3, dst3)


def _post(a0, a1, y0, y1, dinv, bias, prev, h, relu):
  RT = 1024
  has_prev = prev is not None

  def body(*refs):
    if has_prev:
      a0r, a1r, y0r, y1r, dr, br, pr, o_ref = refs
    else:
      a0r, a1r, y0r, y1r, dr, br, o_ref = refs
    d = dr[...]
    o0 = d * (a0r[...] + y0r[...])
    o1 = d * (a1r[...] + y1r[...])
    o = jnp.concatenate([o0, o1], axis=1) + br[...]
    if has_prev:
      o = o + pr[...]
    if relu:
      o = jnp.maximum(o, 0.0)
    o_ref[...] = o

  in_specs = [
      pl.BlockSpec((RT, h), lambda i: (i, 0)),
      pl.BlockSpec((RT, h), lambda i: (i, 0)),
      pl.BlockSpec((RT, h), lambda i: (i, 0)),
      pl.BlockSpec((RT, h), lambda i: (i, 0)),
      pl.BlockSpec((RT, 1), lambda i: (i, 0)),
      pl.BlockSpec((1, 2 * h), lambda i: (0, 0)),
  ]
  args = [a0, a1, y0, y1, dinv, bias]
  if has_prev:
    in_specs.append(pl.BlockSpec((RT, 2 * h), lambda i: (i, 0)))
    args.append(prev)
  return pl.pallas_call(
      body,
      grid=(NP // RT,),
      in_specs=in_specs,
      out_specs=pl.BlockSpec((RT, 2 * h), lambda i: (i, 0)),
      out_shape=jax.ShapeDtypeStruct((NP, 2 * h), jnp.float32),
  )(*args)


def _fused(a0, a1, y0, y1, dinv, bias, prev, oldpieces, wnext, h, relu):
  RT = 512
  n_old = len(oldpieces)
  K = 2 * h
  has_prev = prev is not None

  def body(*refs):
    i = 0
    a0r, a1r, y0r, y1r, dr, br = refs[:6]
    i = 6
    if has_prev:
      pr = refs[i]
      i += 1
    old = refs[i:i + n_old]
    w_ref = refs[i + n_old]
    xb_ref, z0_ref, z1_ref = refs[i + n_old + 1:]
    d = dr[...]
    o0 = d * (a0r[...] + y0r[...])
    o1 = d * (a1r[...] + y1r[...])
    xb = jnp.concatenate([o0, o1], axis=1) + br[...]
    if has_prev:
      xb = xb + pr[...]
    if relu:
      xb = jnp.maximum(xb, 0.0)
    xb_ref[...] = xb
    z = jnp.dot(xb, w_ref[pl.ds(n_old * K, K), :],
                preferred_element_type=jnp.float32)
    for p in range(n_old):
      z = z + jnp.dot(old[p][...], w_ref[pl.ds(p * K, K), :],
                      preferred_element_type=jnp.float32)
    z = z * d
    z0_ref[...] = z[:, :h]
    z1_ref[...] = z[:, h:]

  in_specs = [
      pl.BlockSpec((RT, h), lambda i: (i, 0)),
      pl.BlockSpec((RT, h), lambda i: (i, 0)),
      pl.BlockSpec((RT, h), lambda i: (i, 0)),
      pl.BlockSpec((RT, h), lambda i: (i, 0)),
      pl.BlockSpec((RT, 1), lambda i: (i, 0)),
      pl.BlockSpec((1, K), lambda i: (0, 0)),
  ]
  args = [a0, a1, y0, y1, dinv, bias]
  if has_prev:
    in_specs.append(pl.BlockSpec((RT, K), lambda i: (i, 0)))
    args.append(prev)
  in_specs += [pl.BlockSpec((RT, K), lambda i: (i, 0))] * n_old
  args += list(oldpieces)
  in_specs.append(pl.BlockSpec(((n_old + 1) * K, K), lambda i: (0, 0)))
  args.append(wnext)
  return pl.pallas_call(
      body,
      grid=(NP // RT,),
      in_specs=in_specs,
      out_specs=[pl.BlockSpec((RT, K), lambda i: (i, 0))]
      + [pl.BlockSpec((RT, h), lambda i: (i, 0))] * 2,
      out_shape=[jax.ShapeDtypeStruct((NP, K), jnp.float32)]
      + [jax.ShapeDtypeStruct((NP, h), jnp.float32)] * 2,
  )(*args)


def kernel(x, edge_index, Ws, bs):
  xp = jnp.pad(x.astype(jnp.float32), ((0, NP - N), (0, 0)))
  src = edge_index[0].astype(jnp.int32)
  dst = edge_index[1].astype(jnp.int32)
  pad = jnp.full((EP - E,), NP - 1, jnp.int32)
  src = jnp.concatenate([src, pad])
  dst = jnp.concatenate([dst, pad])
  src3 = src.reshape(NTILE, NCHUNK, CH)
  dst3 = dst.reshape(NTILE, NCHUNK, CH)
  dst32 = dst.reshape(32, EP // 32 // 128, 128)

  degp = _deg_partials(dst32)
  dinv = _dinv(degp.T)

  dlast = Ws[12].shape[1]
  wlast = jnp.pad(Ws[12], ((0, 0), (0, 256 - dlast)))
  blast = jnp.pad(bs[12], (0, 256 - dlast))
  h = 128

  y0, y1 = _pre(xp, Ws[0], dinv, h)
  a0, a1 = _agg(y0, y1, src3, dst3, h)
  xblk, y0, y1 = _fused(a0, a1, y0, y1, dinv, bs[0].reshape(1, 256),
                        None, [], Ws[1], h, False)
  pieces = [xblk]
  prev = xblk
  for i in range(5):
    a0, a1 = _agg(y0, y1, src3, dst3, h)
    wnext = Ws[2 + i] if i < 4 else Ws[6]
    xnew, y0, y1 = _fused(a0, a1, y0, y1, dinv, bs[1 + i].reshape(1, 256),
                          prev, pieces, wnext, h, True)
    pieces = pieces + [xnew]
    prev = xnew
  a0, a1 = _agg(y0, y1, src3, dst3, h)
  g, y0, y1 = _fused(a0, a1, y0, y1, dinv, bs[6].reshape(1, 256),
                     None, [], Ws[7], h, False)
  pieces = [g]
  prev = g
  for i in range(5):
    a0, a1 = _agg(y0, y1, src3, dst3, h)
    wnext = Ws[8 + i] if i < 4 else wlast
    xnew, y0, y1 = _fused(a0, a1, y0, y1, dinv, bs[7 + i].reshape(1, 256),
                          prev, pieces, wnext, h, True)
    pieces = pieces + [xnew]
    prev = xnew
  a0, a1 = _agg(y0, y1, src3, dst3, h)
  out = _post(a0, a1, y0, y1, dinv, blast.reshape(1, 256), None, h, False)
  return out[:N, :dlast]
